# baseline (speedup 1.0000x reference)
import jax
import jax.numpy as jnp
from jax import lax
from jax.experimental import pallas as pl
from jax.experimental.pallas import tpu as pltpu
from jax.experimental.pallas import tpu_sc as plsc

_TIMESTEPS = 1000
_NROWS = 2048
_W = 256
_NWORKERS = 32
_ROWS_PER_W = _NROWS // _NWORKERS
_HOLO = 1024
_LANES = 16


def _build_holo_row(buf, r, hbuf, lane_f32):
    for g in range(_W // _LANES):
        v = buf[r, pl.ds(g * _LANES, _LANES)]
        q0 = (v * 1000.0).astype(jnp.int32) - 1
        q = jnp.where(q0 < 0, _TIMESTEPS - 1, q0)
        valid = v != 0.0
        _, winner = plsc.scan_count(q, mask=valid)
        val = jnp.float32(g * _LANES) + lane_f32
        plsc.store_scatter(hbuf, [q], val, mask=winner)


def _sc_loss_kernel(rec_hbm, tgt_hbm, zero_hbm, out_hbm, rbuf, tbuf,
                    har, hat, hbr, hbt, accv, sem_r, sem_t, sem_za, sem_zb):
    wid = lax.axis_index("c") * 16 + lax.axis_index("s")
    base = wid * _ROWS_PER_W

    cp_r = pltpu.make_async_copy(rec_hbm.at[pl.ds(base, _ROWS_PER_W)],
                                 rbuf, sem_r)
    cp_t = pltpu.make_async_copy(tgt_hbm.at[pl.ds(base, _ROWS_PER_W)],
                                 tbuf, sem_t)
    cp_r.start()
    cp_t.start()

    lane_f32 = lax.iota(jnp.int32, _LANES).astype(jnp.float32)

    cp_r.wait()
    cp_t.wait()

    def scan_pair(hx, hy, accs, a0, a1):
        accs = list(accs)
        for j in range(63):
            sl = pl.ds(j * _LANES, _LANES)
            d = hx[sl] - hy[sl]
            k = a0 if j % 2 == 0 else a1
            accs[k] = accs[k] + d * d
        return tuple(accs)

    def row_body(i, accs):
        r = i * 2
        pltpu.make_async_copy(zero_hbm, har, sem_za).wait()
        pltpu.make_async_copy(zero_hbm, hat, sem_za).wait()
        _build_holo_row(rbuf, r, har, lane_f32)
        _build_holo_row(tbuf, r, hat, lane_f32)
        accs = scan_pair(har, hat, accs, 0, 1)
        pltpu.make_async_copy(zero_hbm, har, sem_za).start()
        pltpu.make_async_copy(zero_hbm, hat, sem_za).start()

        pltpu.make_async_copy(zero_hbm, hbr, sem_zb).wait()
        pltpu.make_async_copy(zero_hbm, hbt, sem_zb).wait()
        _build_holo_row(rbuf, r + 1, hbr, lane_f32)
        _build_holo_row(tbuf, r + 1, hbt, lane_f32)
        accs = scan_pair(hbr, hbt, accs, 2, 3)
        pltpu.make_async_copy(zero_hbm, hbr, sem_zb).start()
        pltpu.make_async_copy(zero_hbm, hbt, sem_zb).start()
        return accs

    pltpu.make_async_copy(zero_hbm, har, sem_za).start()
    pltpu.make_async_copy(zero_hbm, hat, sem_za).start()
    pltpu.make_async_copy(zero_hbm, hbr, sem_zb).start()
    pltpu.make_async_copy(zero_hbm, hbt, sem_zb).start()

    zero4 = (jnp.zeros((_LANES,), jnp.float32),) * 4
    accs = lax.fori_loop(0, _ROWS_PER_W // 2, row_body, zero4)

    pltpu.make_async_copy(zero_hbm, har, sem_za).wait()
    pltpu.make_async_copy(zero_hbm, hat, sem_za).wait()
    pltpu.make_async_copy(zero_hbm, hbr, sem_zb).wait()
    pltpu.make_async_copy(zero_hbm, hbt, sem_zb).wait()

    accv[...] = (accs[0] + accs[1]) + (accs[2] + accs[3])
    pltpu.sync_copy(accv, out_hbm.at[wid])


@jax.jit
def kernel(reconstructed_image, target_image):
    rec = jnp.reshape(reconstructed_image, (_NROWS, _W))
    tgt = jnp.reshape(target_image, (_NROWS, _W))
    zeros = jnp.zeros((_HOLO,), jnp.float32)

    mesh = plsc.VectorSubcoreMesh(core_axis_name="c", subcore_axis_name="s")
    partials = pl.kernel(
        _sc_loss_kernel,
        mesh=mesh,
        compiler_params=pltpu.CompilerParams(needs_layout_passes=False),
        out_type=jax.ShapeDtypeStruct((_NWORKERS, _LANES), jnp.float32),
        scratch_types=[
            pltpu.VMEM((_ROWS_PER_W, _W), jnp.float32),
            pltpu.VMEM((_ROWS_PER_W, _W), jnp.float32),
            pltpu.VMEM((_HOLO,), jnp.float32),
            pltpu.VMEM((_HOLO,), jnp.float32),
            pltpu.VMEM((_HOLO,), jnp.float32),
            pltpu.VMEM((_HOLO,), jnp.float32),
            pltpu.VMEM((_LANES,), jnp.float32),
            pltpu.SemaphoreType.DMA,
            pltpu.SemaphoreType.DMA,
            pltpu.SemaphoreType.DMA,
            pltpu.SemaphoreType.DMA,
        ],
    )(rec, tgt, zeros)

    denom = jnp.float32(8 * 1 * 256 * _TIMESTEPS)
    return jnp.sum(partials) / denom


# speedup vs baseline: 2.6428x; 2.6428x over previous
import jax
import jax.numpy as jnp
from jax import lax
from jax.experimental import pallas as pl
from jax.experimental.pallas import tpu as pltpu
from jax.experimental.pallas import tpu_sc as plsc

_TIMESTEPS = 1000
_NROWS = 2048
_W = 256
_NWORKERS = 32
_ROWS_PER_W = _NROWS // _NWORKERS
_HOLO = 1024
_LANES = 16


def _build_holo_row(buf, r, hbuf, lane_f32):
    for g in range(_W // _LANES):
        v = buf[r, pl.ds(g * _LANES, _LANES)]
        q0 = (v * 1000.0).astype(jnp.int32) - 1
        q = jnp.where(q0 < 0, _TIMESTEPS - 1, q0)
        valid = v != 0.0
        _, winner = plsc.scan_count(q, mask=valid)
        val = jnp.float32(g * _LANES) + lane_f32
        plsc.store_scatter(hbuf, [q], val, mask=winner)


def _sc_loss_kernel(rec_hbm, tgt_hbm, out_hbm, rbuf, tbuf, hr, ht,
                    accv, sem_r, sem_t):
    wid = lax.axis_index("c") * 16 + lax.axis_index("s")
    base = wid * _ROWS_PER_W

    cp_r = pltpu.make_async_copy(rec_hbm.at[pl.ds(base, _ROWS_PER_W)],
                                 rbuf, sem_r)
    cp_t = pltpu.make_async_copy(tgt_hbm.at[pl.ds(base, _ROWS_PER_W)],
                                 tbuf, sem_t)
    cp_r.start()
    cp_t.start()

    lane_f32 = lax.iota(jnp.int32, _LANES).astype(jnp.float32)
    zf = jnp.zeros((_LANES,), jnp.float32)

    @plsc.parallel_loop(0, _HOLO // _LANES)
    def _(j):
        hr[pl.ds(j * _LANES, _LANES)] = zf
        ht[pl.ds(j * _LANES, _LANES)] = zf

    cp_r.wait()
    cp_t.wait()

    def row_body(r, accs):
        _build_holo_row(rbuf, r, hr, lane_f32)
        _build_holo_row(tbuf, r, ht, lane_f32)

        @plsc.parallel_loop(0, 63, unroll=4, carry=tuple(accs))
        def accs(j, a):
            sl = pl.ds(j * _LANES, _LANES)
            d = hr[sl] - ht[sl]
            hr[sl] = zf
            ht[sl] = zf
            return a[1:] + (a[0] + d * d,)

        return accs

    zero4 = (jnp.zeros((_LANES,), jnp.float32),) * 4
    accs = lax.fori_loop(0, _ROWS_PER_W, row_body, zero4)
    accv[...] = (accs[0] + accs[1]) + (accs[2] + accs[3])
    pltpu.sync_copy(accv, out_hbm.at[wid])


@jax.jit
def kernel(reconstructed_image, target_image):
    rec = jnp.reshape(reconstructed_image, (_NROWS, _W))
    tgt = jnp.reshape(target_image, (_NROWS, _W))

    mesh = plsc.VectorSubcoreMesh(core_axis_name="c", subcore_axis_name="s")
    partials = pl.kernel(
        _sc_loss_kernel,
        mesh=mesh,
        compiler_params=pltpu.CompilerParams(needs_layout_passes=False),
        out_type=jax.ShapeDtypeStruct((_NWORKERS, _LANES), jnp.float32),
        scratch_types=[
            pltpu.VMEM((_ROWS_PER_W, _W), jnp.float32),
            pltpu.VMEM((_ROWS_PER_W, _W), jnp.float32),
            pltpu.VMEM((_HOLO,), jnp.float32),
            pltpu.VMEM((_HOLO,), jnp.float32),
            pltpu.VMEM((_LANES,), jnp.float32),
            pltpu.SemaphoreType.DMA,
            pltpu.SemaphoreType.DMA,
        ],
    )(rec, tgt)

    denom = jnp.float32(8 * 1 * 256 * _TIMESTEPS)
    return jnp.sum(partials) / denom
